# Initial kernel scaffold; baseline (speedup 1.0000x reference)
import jax
import jax.numpy as jnp
from jax.experimental import pallas as pl


def kernel(node_feats, edge_feats, edge_index, W_src_gate, b_src_gate, W_dst_gate, b_dst_gate, W_edge_gate, b_edge_gate, W_dst_update, b_dst_update, W_src_update, b_src_update, bn_nodes_gamma, bn_nodes_beta, bn_edges_gamma, bn_edges_beta):
    raise NotImplementedError("write your pallas kernel here")


# speedup vs baseline: 1.0096x; 1.0096x over previous
import functools

import jax
import jax.numpy as jnp
from jax.experimental import pallas as pl


def _node_pre_body(x_ref, wsg_ref, wdg_ref, wdu_ref, wsu_ref,
                   esrc_ref, edst_ref, bh_ref, cx_ref):
    x = x_ref[...]
    esrc_ref[...] = x @ wsg_ref[...]
    edst_ref[...] = x @ wdg_ref[...]
    bh_ref[...] = x @ wdu_ref[...]
    cx_ref[...] = x @ wsu_ref[...]


def _edge_gate_body(ef_ref, weg_ref, out_ref):
    out_ref[...] = ef_ref[...] @ weg_ref[...]


def kernel(node_feats, edge_feats, edge_index, W_src_gate, b_src_gate,
           W_dst_gate, b_dst_gate, W_edge_gate, b_edge_gate,
           W_dst_update, b_dst_update, W_src_update, b_src_update,
           bn_nodes_gamma, bn_nodes_beta, bn_edges_gamma, bn_edges_beta):
    N, D = node_feats.shape
    E = edge_feats.shape[0]
    src = edge_index[0]
    dst = edge_index[1]

    nb = 1000
    e_src, e_dst, Bh, Cx = pl.pallas_call(
        _node_pre_body,
        grid=(N // nb,),
        in_specs=[
            pl.BlockSpec((nb, D), lambda i: (i, 0)),
            pl.BlockSpec((D, D), lambda i: (0, 0)),
            pl.BlockSpec((D, D), lambda i: (0, 0)),
            pl.BlockSpec((D, D), lambda i: (0, 0)),
            pl.BlockSpec((D, D), lambda i: (0, 0)),
        ],
        out_specs=[pl.BlockSpec((nb, D), lambda i: (i, 0))] * 4,
        out_shape=[jax.ShapeDtypeStruct((N, D), jnp.float32)] * 4,
    )(node_feats, W_src_gate, W_dst_gate, W_dst_update, W_src_update)
    e_src = e_src + b_src_gate
    e_dst = e_dst + b_dst_gate
    Bh = Bh + b_dst_update
    Cx = Cx + b_src_update

    eb = 4000
    Eg = pl.pallas_call(
        _edge_gate_body,
        grid=(E // eb,),
        in_specs=[
            pl.BlockSpec((eb, D), lambda i: (i, 0)),
            pl.BlockSpec((D, D), lambda i: (0, 0)),
        ],
        out_specs=pl.BlockSpec((eb, D), lambda i: (i, 0)),
        out_shape=jax.ShapeDtypeStruct((E, D), jnp.float32),
    )(edge_feats, W_edge_gate)

    m = jnp.take(e_src, src, axis=0) + jnp.take(e_dst, dst, axis=0) + Eg + b_edge_gate
    sigma = jax.nn.sigmoid(m)
    sum_sigma_h = jax.ops.segment_sum(sigma * jnp.take(Bh, src, axis=0), dst,
                                      num_segments=N)
    sum_sigma = jax.ops.segment_sum(sigma, dst, num_segments=N)
    h = sum_sigma_h / (sum_sigma + 1e-6)

    def _bn_silu(v, gamma, beta, res):
        mu = jnp.mean(v, axis=0)
        var = jnp.var(v, axis=0)
        t = gamma * (v - mu) / jnp.sqrt(var + 1e-5) + beta
        return res + t * jax.nn.sigmoid(t)

    x = _bn_silu(Cx + h, bn_nodes_gamma, bn_nodes_beta, node_feats)
    y = _bn_silu(m, bn_edges_gamma, bn_edges_beta, edge_feats)
    return (x, y)


# speedup vs baseline: 2.7373x; 2.7113x over previous
import functools

import jax
import jax.numpy as jnp
from jax import lax
from jax.experimental import pallas as pl
from jax.experimental.pallas import tpu as pltpu
from jax.experimental.pallas import tpu_sc as plsc

N = 10000
E = 320000
D = 128
H = 64
NT = 16
ET = E // NT
C = 80
NCHUNK = ET // C
SPLIT = 9992
TAIL = N - SPLIT


def _node_pre_body(x_ref, wsg, wdg, wdu, wsu, bdu, bsu,
                   tsrc0, tsrc1, edst, cx):
    x = x_ref[...]
    t_es = x @ wsg[...]
    t_bh = x @ wdu[...] + bdu[...]
    tsrc0[...] = jnp.concatenate([t_es[:, :H], t_bh[:, :H]], axis=1)
    tsrc1[...] = jnp.concatenate([t_es[:, H:], t_bh[:, H:]], axis=1)
    edst[...] = x @ wdg[...]
    cx[...] = x @ wsu[...] + bsu[...]


def _gate_body(ef, weg, bias, eg0, eg1):
    t = ef[...] @ weg[...] + bias[...]
    eg0[...] = t[:, :H]
    eg1[...] = t[:, H:]


def _sc_body(src_hbm, dst_hbm, ts0, ts1, td, eg0, eg1, zer,
             m0_hbm, m1_hbm, acc0_hbm, acc1_hbm, tails_hbm, stats_hbm,
             src_v, dst_v, ilo_v, ihi_v, a_v, d_v, e_v, comb_v,
             stat_v, acc_sh, acc2_sh,
             sem0, sem1, sem2):
    c = lax.axis_index("c")
    s = lax.axis_index("s")

    @pl.when(s == 0)
    def _():
        pltpu.sync_copy(zer.at[pl.ds(0, SPLIT)], acc_sh.at[pl.ds(0, SPLIT)])
        pltpu.sync_copy(zer.at[pl.ds(0, TAIL)], acc2_sh.at[pl.ds(0, TAIL)])

    plsc.subcore_barrier()

    def run_half(ts, eg, off, m_hbm):
        tile_base = s * ET

        def chunk_body(i, carry):
            base = tile_base + i * C
            pltpu.sync_copy(src_hbm.at[pl.ds(base, C)], src_v)
            pltpu.sync_copy(dst_hbm.at[pl.ds(base, C)], dst_v)
            cp0 = pltpu.async_copy(ts.at[src_v], a_v, sem0)
            cp1 = pltpu.async_copy(td.at[dst_v], d_v, sem1)
            cp2 = pltpu.async_copy(eg.at[pl.ds(base, C)], e_v, sem2)
            for j in range(C // 16):
                sl = pl.ds(j * 16, 16)
                dv = dst_v[sl]
                tl = dv >= SPLIT
                ilo_v[sl] = jnp.where(tl, SPLIT, dv)
                ihi_v[sl] = jnp.where(tl, dv - SPLIT, TAIL)
            cp0.wait()
            cp1.wait()
            cp2.wait()

            def row_body(r, rc):
                acc = list(rc)
                for k in range(4):
                    sl = pl.ds(k * 16, 16)
                    slh = pl.ds(off + k * 16, 16)
                    m = a_v[r, sl] + d_v[r, slh] + e_v[r, sl]
                    e_v[r, sl] = m
                    sig = 1.0 / (1.0 + jnp.exp(-m))
                    comb_v[r, sl] = sig * a_v[r, pl.ds(H + k * 16, 16)]
                    comb_v[r, pl.ds(H + k * 16, 16)] = sig
                    acc[k] = acc[k] + m
                    acc[4 + k] = acc[4 + k] + m * m
                return tuple(acc)

            carry = lax.fori_loop(0, C, row_body, carry)
            pltpu.sync_copy(e_v, m_hbm.at[pl.ds(base, C)])
            pltpu.sync_copy(comb_v, acc_sh.at[ilo_v], add=True)
            pltpu.sync_copy(comb_v, acc2_sh.at[ihi_v], add=True)
            return carry

        z = jnp.zeros((16,), jnp.float32)
        carry = lax.fori_loop(0, NCHUNK, chunk_body, (z,) * 8)
        for k in range(4):
            stat_v[pl.ds(k * 16, 16)] = carry[k]
            stat_v[pl.ds(H + k * 16, 16)] = carry[4 + k]
        pltpu.sync_copy(stat_v, stats_hbm.at[c, s])

    @pl.when(c == 0)
    def _():
        run_half(ts0, eg0, 0, m0_hbm)

    @pl.when(c == 1)
    def _():
        run_half(ts1, eg1, H, m1_hbm)

    plsc.subcore_barrier()

    @pl.when((s == 0) & (c == 0))
    def _():
        pltpu.sync_copy(acc_sh.at[pl.ds(0, SPLIT)], acc0_hbm)
        pltpu.sync_copy(acc2_sh.at[pl.ds(0, TAIL)], tails_hbm.at[0])

    @pl.when((s == 0) & (c == 1))
    def _():
        pltpu.sync_copy(acc_sh.at[pl.ds(0, SPLIT)], acc1_hbm)
        pltpu.sync_copy(acc2_sh.at[pl.ds(0, TAIL)], tails_hbm.at[1])


def _edge_epi_body(m0, m1, ef, stats, gamma, beta, y):
    st = stats[...]
    red = jnp.sum(st, axis=1)
    sum_m = jnp.concatenate([red[0:1, 0:H], red[1:2, 0:H]], axis=1)
    sum_q = jnp.concatenate([red[0:1, H:], red[1:2, H:]], axis=1)
    mu = sum_m * (1.0 / E)
    var = sum_q * (1.0 / E) - mu * mu
    m = jnp.concatenate([m0[...], m1[...]], axis=1)
    t = gamma[...] * (m - mu) * lax.rsqrt(var + 1e-5) + beta[...]
    y[...] = ef[...] + t * (1.0 / (1.0 + jnp.exp(-t)))


def _node_epi_body(acc0, acc1, tails, cx, nf, gamma, beta, x):
    t = tails[...]
    a0 = jnp.concatenate([acc0[...], t[0]], axis=0)
    a1 = jnp.concatenate([acc1[...], t[1]], axis=0)
    num = jnp.concatenate([a0[:, :H], a1[:, :H]], axis=1)
    den = jnp.concatenate([a0[:, H:], a1[:, H:]], axis=1)
    v = cx[...] + num / (den + 1e-6)
    mu = jnp.mean(v, axis=0, keepdims=True)
    var = jnp.mean(v * v, axis=0, keepdims=True) - mu * mu
    t = gamma[...] * (v - mu) * lax.rsqrt(var + 1e-5) + beta[...]
    x[...] = nf[...] + t * (1.0 / (1.0 + jnp.exp(-t)))


def kernel(node_feats, edge_feats, edge_index, W_src_gate, b_src_gate,
           W_dst_gate, b_dst_gate, W_edge_gate, b_edge_gate,
           W_dst_update, b_dst_update, W_src_update, b_src_update,
           bn_nodes_gamma, bn_nodes_beta, bn_edges_gamma, bn_edges_beta):
    src = edge_index[0]
    dst = edge_index[1]
    f32 = jnp.float32

    nb = 1000
    bdu = b_dst_update.reshape(1, D)
    bsu = b_src_update.reshape(1, D)
    ts0, ts1, e_dst, Cx = pl.pallas_call(
        _node_pre_body,
        grid=(N // nb,),
        in_specs=[
            pl.BlockSpec((nb, D), lambda i: (i, 0)),
            pl.BlockSpec((D, D), lambda i: (0, 0)),
            pl.BlockSpec((D, D), lambda i: (0, 0)),
            pl.BlockSpec((D, D), lambda i: (0, 0)),
            pl.BlockSpec((D, D), lambda i: (0, 0)),
            pl.BlockSpec((1, D), lambda i: (0, 0)),
            pl.BlockSpec((1, D), lambda i: (0, 0)),
        ],
        out_specs=[pl.BlockSpec((nb, D), lambda i: (i, 0))] * 4,
        out_shape=[jax.ShapeDtypeStruct((N, D), f32)] * 4,
    )(node_feats, W_src_gate, W_dst_gate, W_dst_update, W_src_update,
      bdu, bsu)

    eb = 4000
    gate_bias = (b_edge_gate + b_src_gate + b_dst_gate).reshape(1, D)
    eg0, eg1 = pl.pallas_call(
        _gate_body,
        grid=(E // eb,),
        in_specs=[
            pl.BlockSpec((eb, D), lambda i: (i, 0)),
            pl.BlockSpec((D, D), lambda i: (0, 0)),
            pl.BlockSpec((1, D), lambda i: (0, 0)),
        ],
        out_specs=[pl.BlockSpec((eb, H), lambda i: (i, 0))] * 2,
        out_shape=[jax.ShapeDtypeStruct((E, H), f32)] * 2,
    )(edge_feats, W_edge_gate, gate_bias)

    zer = jnp.zeros((N, D), f32)

    mesh = plsc.VectorSubcoreMesh(core_axis_name="c", subcore_axis_name="s")
    sc_fn = pl.kernel(
        _sc_body,
        out_type=[
            jax.ShapeDtypeStruct((E, H), f32),
            jax.ShapeDtypeStruct((E, H), f32),
            jax.ShapeDtypeStruct((SPLIT, D), f32),
            jax.ShapeDtypeStruct((SPLIT, D), f32),
            jax.ShapeDtypeStruct((2, TAIL, D), f32),
            jax.ShapeDtypeStruct((2, NT, D), f32),
        ],
        mesh=mesh,
        scratch_types=[
            pltpu.VMEM((C,), jnp.int32),
            pltpu.VMEM((C,), jnp.int32),
            pltpu.VMEM((C,), jnp.int32),
            pltpu.VMEM((C,), jnp.int32),
            pltpu.VMEM((C, D), f32),
            pltpu.VMEM((C, D), f32),
            pltpu.VMEM((C, H), f32),
            pltpu.VMEM((C, D), f32),
            pltpu.VMEM((D,), f32),
            pltpu.VMEM_SHARED((SPLIT + 1, D), f32),
            pltpu.VMEM_SHARED((TAIL + 1, D), f32),
            pltpu.SemaphoreType.DMA,
            pltpu.SemaphoreType.DMA,
            pltpu.SemaphoreType.DMA,
        ],
    )
    m0, m1, acc0, acc1, tails, stats = sc_fn(src, dst, ts0, ts1, e_dst,
                                             eg0, eg1, zer)

    y = pl.pallas_call(
        _edge_epi_body,
        grid=(E // eb,),
        in_specs=[
            pl.BlockSpec((eb, H), lambda i: (i, 0)),
            pl.BlockSpec((eb, H), lambda i: (i, 0)),
            pl.BlockSpec((eb, D), lambda i: (i, 0)),
            pl.BlockSpec((2, NT, D), lambda i: (0, 0, 0)),
            pl.BlockSpec((1, D), lambda i: (0, 0)),
            pl.BlockSpec((1, D), lambda i: (0, 0)),
        ],
        out_specs=pl.BlockSpec((eb, D), lambda i: (i, 0)),
        out_shape=jax.ShapeDtypeStruct((E, D), f32),
    )(m0, m1, edge_feats, stats, bn_edges_gamma.reshape(1, D),
      bn_edges_beta.reshape(1, D))

    x = pl.pallas_call(
        _node_epi_body,
        in_specs=[pl.BlockSpec((SPLIT, D), lambda: (0, 0))] * 2 +
                 [pl.BlockSpec((2, TAIL, D), lambda: (0, 0, 0))] +
                 [pl.BlockSpec((N, D), lambda: (0, 0))] * 2 +
                 [pl.BlockSpec((1, D), lambda: (0, 0))] * 2,
        out_specs=pl.BlockSpec((N, D), lambda: (0, 0)),
        out_shape=jax.ShapeDtypeStruct((N, D), f32),
    )(acc0, acc1, tails, Cx, node_feats, bn_nodes_gamma.reshape(1, D),
      bn_nodes_beta.reshape(1, D))

    return (x, y)


# speedup vs baseline: 3.2710x; 1.1950x over previous
import functools

import jax
import jax.numpy as jnp
from jax import lax
from jax.experimental import pallas as pl
from jax.experimental.pallas import tpu as pltpu
from jax.experimental.pallas import tpu_sc as plsc

N = 10000
E = 320000
D = 128
H = 64
NT = 16
ET = E // NT
C = 32
NCHUNK = ET // C
SPLIT = 9992
TAIL = N - SPLIT


def _node_pre_body(x_ref, wsg, wdg, wdu, wsu, bdu, bsu,
                   tsrc0, tsrc1, edst, cx):
    x = x_ref[...]
    t_es = x @ wsg[...]
    t_bh = x @ wdu[...] + bdu[...]
    tsrc0[...] = jnp.concatenate([t_es[:, :H], t_bh[:, :H]], axis=1)
    tsrc1[...] = jnp.concatenate([t_es[:, H:], t_bh[:, H:]], axis=1)
    edst[...] = x @ wdg[...]
    cx[...] = x @ wsu[...] + bsu[...]


def _gate_body(ef, weg, bias, eg0, eg1):
    t = ef[...] @ weg[...] + bias[...]
    eg0[...] = t[:, :H]
    eg1[...] = t[:, H:]


def _sc_body(src_hbm, dst_hbm, ts0, ts1, td, eg0, eg1, zer,
             m0_hbm, m1_hbm, acc0_hbm, acc1_hbm, tails_hbm, stats_hbm,
             sq0, sq1, tq0, tq1, av0, av1, dv0, dv1, ev0, ev1, cb0, cb1,
             lo0, lo1, hi0, hi1, stat_v, acc_sh, acc2_sh,
             ss0, ss1, st0, st1, sa0, sa1, sd0, sd1, se0, se1,
             sm0, sm1, sx0, sx1, sy0, sy1):
    c = lax.axis_index("c")
    s = lax.axis_index("s")
    srcq = [sq0, sq1]
    dstq = [tq0, tq1]
    av = [av0, av1]
    dvv = [dv0, dv1]
    ev = [ev0, ev1]
    cbv = [cb0, cb1]
    lov = [lo0, lo1]
    hiv = [hi0, hi1]
    sem_s = [ss0, ss1]
    sem_t = [st0, st1]
    sem_a = [sa0, sa1]
    sem_d = [sd0, sd1]
    sem_e = [se0, se1]
    sem_m = [sm0, sm1]
    sem_x = [sx0, sx1]
    sem_y = [sy0, sy1]

    @pl.when(s == 0)
    def _():
        pltpu.sync_copy(zer.at[pl.ds(0, SPLIT)], acc_sh.at[pl.ds(0, SPLIT)])
        pltpu.sync_copy(zer.at[pl.ds(0, TAIL)], acc2_sh.at[pl.ds(0, TAIL)])

    plsc.subcore_barrier()

    def run_half(ts, eg, off, m_hbm):
        tile_base = s * ET

        def cbase(ci):
            return tile_base + jnp.minimum(ci * C, ET - C)

        def idx_issue(ci, q):
            b = cbase(ci)
            pltpu.async_copy(src_hbm.at[pl.ds(b, C)], srcq[q], sem_s[q])
            pltpu.async_copy(dst_hbm.at[pl.ds(b, C)], dstq[q], sem_t[q])

        def idx_wait(q):
            pltpu.make_async_copy(src_hbm.at[pl.ds(0, C)], srcq[q],
                                  sem_s[q]).wait()
            pltpu.make_async_copy(dst_hbm.at[pl.ds(0, C)], dstq[q],
                                  sem_t[q]).wait()

        def gather_issue(ci, g, q):
            b = cbase(ci)
            pltpu.async_copy(ts.at[srcq[q]], av[g], sem_a[g])
            pltpu.async_copy(td.at[dstq[q]], dvv[g], sem_d[g])
            pltpu.async_copy(eg.at[pl.ds(b, C)], ev[g], sem_e[g])

        def gather_wait(g, q):
            pltpu.make_async_copy(ts.at[srcq[q]], av[g], sem_a[g]).wait()
            pltpu.make_async_copy(td.at[dstq[q]], dvv[g], sem_d[g]).wait()
            pltpu.make_async_copy(eg.at[pl.ds(0, C)], ev[g], sem_e[g]).wait()

        def ilo_compute(g, q):
            for j in range(C // 16):
                sl = pl.ds(j * 16, 16)
                dj = dstq[q][sl]
                tl = dj >= SPLIT
                lov[g][sl] = jnp.where(tl, SPLIT, dj)
                hiv[g][sl] = jnp.where(tl, dj - SPLIT, TAIL)

        def rows(g, carry):
            def row_body(r, rc):
                acc = list(rc)
                for k in range(4):
                    sl = pl.ds(k * 16, 16)
                    slh = pl.ds(off + k * 16, 16)
                    m = av[g][r, sl] + dvv[g][r, slh] + ev[g][r, sl]
                    ev[g][r, sl] = m
                    sig = 1.0 / (1.0 + jnp.exp(-m))
                    cbv[g][r, sl] = sig * av[g][r, pl.ds(H + k * 16, 16)]
                    cbv[g][r, pl.ds(H + k * 16, 16)] = sig
                    acc[k] = acc[k] + m
                    acc[4 + k] = acc[4 + k] + m * m
                return tuple(acc)

            return lax.fori_loop(0, C, row_body, carry)

        def outs_issue(ci, g):
            b = cbase(ci)
            pltpu.async_copy(ev[g], m_hbm.at[pl.ds(b, C)], sem_m[g])
            pltpu.async_copy(cbv[g], acc_sh.at[lov[g]], sem_x[g], add=True)
            pltpu.async_copy(cbv[g], acc2_sh.at[hiv[g]], sem_y[g], add=True)

        def outs_wait(g):
            pltpu.make_async_copy(ev[g], m_hbm.at[pl.ds(0, C)],
                                  sem_m[g]).wait()
            pltpu.make_async_copy(cbv[g], acc_sh.at[lov[g]], sem_x[g]).wait()
            pltpu.make_async_copy(cbv[g], acc2_sh.at[hiv[g]],
                                  sem_y[g]).wait()

        idx_issue(0, 0)
        idx_wait(0)
        gather_issue(0, 1, 0)
        ilo_compute(1, 0)
        idx_issue(1, 1)
        gather_wait(1, 0)
        idx_issue(2, 0)
        z = jnp.zeros((16,), jnp.float32)
        carry = rows(1, (z,) * 8)
        outs_issue(0, 1)
        idx_wait(1)
        gather_issue(1, 0, 1)
        ilo_compute(0, 1)

        def pair_body(i2, carry):
            i = 1 + 2 * i2
            gather_wait(0, 1)
            idx_issue(i + 2, 1)
            carry = rows(0, carry)
            outs_issue(i, 0)
            outs_wait(1)
            idx_wait(0)
            gather_issue(i + 1, 1, 0)
            ilo_compute(1, 0)
            gather_wait(1, 0)
            idx_issue(i + 3, 0)
            carry = rows(1, carry)
            outs_issue(i + 1, 1)
            outs_wait(0)
            idx_wait(1)
            gather_issue(i + 2, 0, 1)
            ilo_compute(0, 1)
            return carry

        carry = lax.fori_loop(0, (NCHUNK - 1) // 2, pair_body, carry)

        outs_wait(1)
        gather_wait(0, 1)
        idx_wait(0)

        for k in range(4):
            stat_v[pl.ds(k * 16, 16)] = carry[k]
            stat_v[pl.ds(H + k * 16, 16)] = carry[4 + k]
        pltpu.sync_copy(stat_v, stats_hbm.at[c, s])

    @pl.when(c == 0)
    def _():
        run_half(ts0, eg0, 0, m0_hbm)

    @pl.when(c == 1)
    def _():
        run_half(ts1, eg1, H, m1_hbm)

    plsc.subcore_barrier()

    @pl.when((s == 0) & (c == 0))
    def _():
        pltpu.sync_copy(acc_sh.at[pl.ds(0, SPLIT)], acc0_hbm)
        pltpu.sync_copy(acc2_sh.at[pl.ds(0, TAIL)], tails_hbm.at[0])

    @pl.when((s == 0) & (c == 1))
    def _():
        pltpu.sync_copy(acc_sh.at[pl.ds(0, SPLIT)], acc1_hbm)
        pltpu.sync_copy(acc2_sh.at[pl.ds(0, TAIL)], tails_hbm.at[1])


def _edge_epi_body(m0, m1, ef, stats, gamma, beta, y):
    st = stats[...]
    red = jnp.sum(st, axis=1)
    sum_m = jnp.concatenate([red[0:1, 0:H], red[1:2, 0:H]], axis=1)
    sum_q = jnp.concatenate([red[0:1, H:], red[1:2, H:]], axis=1)
    mu = sum_m * (1.0 / E)
    var = sum_q * (1.0 / E) - mu * mu
    m = jnp.concatenate([m0[...], m1[...]], axis=1)
    t = gamma[...] * (m - mu) * lax.rsqrt(var + 1e-5) + beta[...]
    y[...] = ef[...] + t * (1.0 / (1.0 + jnp.exp(-t)))


def _node_epi_body(acc0, acc1, tails, cx, nf, gamma, beta, x):
    t = tails[...]
    a0 = jnp.concatenate([acc0[...], t[0]], axis=0)
    a1 = jnp.concatenate([acc1[...], t[1]], axis=0)
    num = jnp.concatenate([a0[:, :H], a1[:, :H]], axis=1)
    den = jnp.concatenate([a0[:, H:], a1[:, H:]], axis=1)
    v = cx[...] + num / (den + 1e-6)
    mu = jnp.mean(v, axis=0, keepdims=True)
    var = jnp.mean(v * v, axis=0, keepdims=True) - mu * mu
    t = gamma[...] * (v - mu) * lax.rsqrt(var + 1e-5) + beta[...]
    x[...] = nf[...] + t * (1.0 / (1.0 + jnp.exp(-t)))


def kernel(node_feats, edge_feats, edge_index, W_src_gate, b_src_gate,
           W_dst_gate, b_dst_gate, W_edge_gate, b_edge_gate,
           W_dst_update, b_dst_update, W_src_update, b_src_update,
           bn_nodes_gamma, bn_nodes_beta, bn_edges_gamma, bn_edges_beta):
    src = edge_index[0]
    dst = edge_index[1]
    f32 = jnp.float32

    nb = 1000
    bdu = b_dst_update.reshape(1, D)
    bsu = b_src_update.reshape(1, D)
    ts0, ts1, e_dst, Cx = pl.pallas_call(
        _node_pre_body,
        grid=(N // nb,),
        in_specs=[
            pl.BlockSpec((nb, D), lambda i: (i, 0)),
            pl.BlockSpec((D, D), lambda i: (0, 0)),
            pl.BlockSpec((D, D), lambda i: (0, 0)),
            pl.BlockSpec((D, D), lambda i: (0, 0)),
            pl.BlockSpec((D, D), lambda i: (0, 0)),
            pl.BlockSpec((1, D), lambda i: (0, 0)),
            pl.BlockSpec((1, D), lambda i: (0, 0)),
        ],
        out_specs=[pl.BlockSpec((nb, D), lambda i: (i, 0))] * 4,
        out_shape=[jax.ShapeDtypeStruct((N, D), f32)] * 4,
    )(node_feats, W_src_gate, W_dst_gate, W_dst_update, W_src_update,
      bdu, bsu)

    eb = 4000
    gate_bias = (b_edge_gate + b_src_gate + b_dst_gate).reshape(1, D)
    eg0, eg1 = pl.pallas_call(
        _gate_body,
        grid=(E // eb,),
        in_specs=[
            pl.BlockSpec((eb, D), lambda i: (i, 0)),
            pl.BlockSpec((D, D), lambda i: (0, 0)),
            pl.BlockSpec((1, D), lambda i: (0, 0)),
        ],
        out_specs=[pl.BlockSpec((eb, H), lambda i: (i, 0))] * 2,
        out_shape=[jax.ShapeDtypeStruct((E, H), f32)] * 2,
    )(edge_feats, W_edge_gate, gate_bias)

    zer = jnp.zeros((N, D), f32)

    mesh = plsc.VectorSubcoreMesh(core_axis_name="c", subcore_axis_name="s")
    sc_fn = pl.kernel(
        _sc_body,
        out_type=[
            jax.ShapeDtypeStruct((E, H), f32),
            jax.ShapeDtypeStruct((E, H), f32),
            jax.ShapeDtypeStruct((SPLIT, D), f32),
            jax.ShapeDtypeStruct((SPLIT, D), f32),
            jax.ShapeDtypeStruct((2, TAIL, D), f32),
            jax.ShapeDtypeStruct((2, NT, D), f32),
        ],
        mesh=mesh,
        scratch_types=(
            [pltpu.VMEM((C,), jnp.int32)] * 4 +
            [pltpu.VMEM((C, D), f32)] * 2 +
            [pltpu.VMEM((C, D), f32)] * 2 +
            [pltpu.VMEM((C, H), f32)] * 2 +
            [pltpu.VMEM((C, D), f32)] * 2 +
            [pltpu.VMEM((C,), jnp.int32)] * 4 +
            [pltpu.VMEM((D,), f32),
             pltpu.VMEM_SHARED((SPLIT + 1, D), f32),
             pltpu.VMEM_SHARED((TAIL + 1, D), f32)] +
            [pltpu.SemaphoreType.DMA] * 16
        ),
    )
    m0, m1, acc0, acc1, tails, stats = sc_fn(src, dst, ts0, ts1, e_dst,
                                             eg0, eg1, zer)

    y = pl.pallas_call(
        _edge_epi_body,
        grid=(E // eb,),
        in_specs=[
            pl.BlockSpec((eb, H), lambda i: (i, 0)),
            pl.BlockSpec((eb, H), lambda i: (i, 0)),
            pl.BlockSpec((eb, D), lambda i: (i, 0)),
            pl.BlockSpec((2, NT, D), lambda i: (0, 0, 0)),
            pl.BlockSpec((1, D), lambda i: (0, 0)),
            pl.BlockSpec((1, D), lambda i: (0, 0)),
        ],
        out_specs=pl.BlockSpec((eb, D), lambda i: (i, 0)),
        out_shape=jax.ShapeDtypeStruct((E, D), f32),
    )(m0, m1, edge_feats, stats, bn_edges_gamma.reshape(1, D),
      bn_edges_beta.reshape(1, D))

    x = pl.pallas_call(
        _node_epi_body,
        in_specs=[pl.BlockSpec((SPLIT, D), lambda: (0, 0))] * 2 +
                 [pl.BlockSpec((2, TAIL, D), lambda: (0, 0, 0))] +
                 [pl.BlockSpec((N, D), lambda: (0, 0))] * 2 +
                 [pl.BlockSpec((1, D), lambda: (0, 0))] * 2,
        out_specs=pl.BlockSpec((N, D), lambda: (0, 0)),
        out_shape=jax.ShapeDtypeStruct((N, D), f32),
    )(acc0, acc1, tails, Cx, node_feats, bn_nodes_gamma.reshape(1, D),
      bn_nodes_beta.reshape(1, D))

    return (x, y)


# speedup vs baseline: 3.3027x; 1.0097x over previous
import functools

import jax
import jax.numpy as jnp
from jax import lax
from jax.experimental import pallas as pl
from jax.experimental.pallas import tpu as pltpu
from jax.experimental.pallas import tpu_sc as plsc

N = 10000
E = 320000
D = 128
H = 64
NT = 16
ET = E // NT
C = 32
NCHUNK = ET // C


def _node_pre_body(x_ref, wsg, wdg, wdu, wsu, bdu, bsu,
                   tsrc0, tsrc1, edst, cx):
    x = x_ref[...]
    t_es = x @ wsg[...]
    t_bh = x @ wdu[...] + bdu[...]
    tsrc0[...] = jnp.concatenate([t_es[:, :H], t_bh[:, :H]], axis=1)
    tsrc1[...] = jnp.concatenate([t_es[:, H:], t_bh[:, H:]], axis=1)
    edst[...] = x @ wdg[...]
    cx[...] = x @ wsu[...] + bsu[...]


def _gate_body(ef, weg, bias, eg0, eg1):
    t = ef[...] @ weg[...] + bias[...]
    eg0[...] = t[:, :H]
    eg1[...] = t[:, H:]


def _sc_body(src_hbm, dst_hbm, ts0, ts1, td, eg0, eg1, zer,
             m0_hbm, m1_hbm, acc0_hbm, acc1_hbm, stats_hbm,
             sq0, sq1, tq0, tq1, av0, av1, dv0, dv1, ev0, ev1, cb0, cb1,
             lo0, lo1, stat_v, acc_sh,
             ss0, ss1, st0, st1, sa0, sa1, sd0, sd1, se0, se1,
             sm0, sm1, sx0, sx1):
    c = lax.axis_index("c")
    s = lax.axis_index("s")
    srcq = [sq0, sq1]
    dstq = [tq0, tq1]
    av = [av0, av1]
    dvv = [dv0, dv1]
    ev = [ev0, ev1]
    cbv = [cb0, cb1]
    lov = [lo0, lo1]
    sem_s = [ss0, ss1]
    sem_t = [st0, st1]
    sem_a = [sa0, sa1]
    sem_d = [sd0, sd1]
    sem_e = [se0, se1]
    sem_m = [sm0, sm1]
    sem_x = [sx0, sx1]

    @pl.when(s == 0)
    def _():
        pltpu.sync_copy(zer, acc_sh)

    plsc.subcore_barrier()

    def run_half(ts, eg, off, m_hbm):
        tile_base = s * ET

        def cbase(ci):
            return tile_base + jnp.minimum(ci * C, ET - C)

        def idx_issue(ci, q):
            b = cbase(ci)
            pltpu.async_copy(src_hbm.at[pl.ds(b, C)], srcq[q], sem_s[q])
            pltpu.async_copy(dst_hbm.at[pl.ds(b, C)], dstq[q], sem_t[q])

        def idx_wait(q):
            pltpu.make_async_copy(src_hbm.at[pl.ds(0, C)], srcq[q],
                                  sem_s[q]).wait()
            pltpu.make_async_copy(dst_hbm.at[pl.ds(0, C)], dstq[q],
                                  sem_t[q]).wait()

        def gather_issue(ci, g, q):
            b = cbase(ci)
            pltpu.async_copy(ts.at[srcq[q]], av[g], sem_a[g])
            pltpu.async_copy(td.at[dstq[q]], dvv[g], sem_d[g])
            pltpu.async_copy(eg.at[pl.ds(b, C)], ev[g], sem_e[g])

        def gather_wait(g, q):
            pltpu.make_async_copy(ts.at[srcq[q]], av[g], sem_a[g]).wait()
            pltpu.make_async_copy(td.at[dstq[q]], dvv[g], sem_d[g]).wait()
            pltpu.make_async_copy(eg.at[pl.ds(0, C)], ev[g], sem_e[g]).wait()

        def ilo_compute(g, q):
            for j in range(C // 16):
                sl = pl.ds(j * 16, 16)
                lov[g][sl] = dstq[q][sl]

        def rows(g, carry):
            def row_body(r, rc):
                acc = list(rc)
                for k in range(4):
                    sl = pl.ds(k * 16, 16)
                    slh = pl.ds(off + k * 16, 16)
                    m = av[g][r, sl] + dvv[g][r, slh] + ev[g][r, sl]
                    ev[g][r, sl] = m
                    sig = 1.0 / (1.0 + jnp.exp(-m))
                    cbv[g][r, sl] = sig * av[g][r, pl.ds(H + k * 16, 16)]
                    cbv[g][r, pl.ds(H + k * 16, 16)] = sig
                    acc[k] = acc[k] + m
                    acc[4 + k] = acc[4 + k] + m * m
                return tuple(acc)

            return lax.fori_loop(0, C, row_body, carry)

        def outs_issue(ci, g):
            b = cbase(ci)
            pltpu.async_copy(ev[g], m_hbm.at[pl.ds(b, C)], sem_m[g])
            pltpu.async_copy(cbv[g], acc_sh.at[lov[g]], sem_x[g], add=True)

        def outs_wait(g):
            pltpu.make_async_copy(ev[g], m_hbm.at[pl.ds(0, C)],
                                  sem_m[g]).wait()
            pltpu.make_async_copy(cbv[g], acc_sh.at[lov[g]], sem_x[g]).wait()

        idx_issue(0, 0)
        idx_wait(0)
        gather_issue(0, 1, 0)
        ilo_compute(1, 0)
        idx_issue(1, 1)
        gather_wait(1, 0)
        idx_issue(2, 0)
        z = jnp.zeros((16,), jnp.float32)
        carry = rows(1, (z,) * 8)
        outs_issue(0, 1)
        idx_wait(1)
        gather_issue(1, 0, 1)
        ilo_compute(0, 1)

        def pair_body(i2, carry):
            i = 1 + 2 * i2
            gather_wait(0, 1)
            idx_issue(i + 2, 1)
            carry = rows(0, carry)
            outs_issue(i, 0)
            outs_wait(1)
            idx_wait(0)
            gather_issue(i + 1, 1, 0)
            ilo_compute(1, 0)
            gather_wait(1, 0)
            idx_issue(i + 3, 0)
            carry = rows(1, carry)
            outs_issue(i + 1, 1)
            outs_wait(0)
            idx_wait(1)
            gather_issue(i + 2, 0, 1)
            ilo_compute(0, 1)
            return carry

        carry = lax.fori_loop(0, (NCHUNK - 1) // 2, pair_body, carry)

        outs_wait(1)
        gather_wait(0, 1)
        idx_wait(0)

        for k in range(4):
            stat_v[pl.ds(k * 16, 16)] = carry[k]
            stat_v[pl.ds(H + k * 16, 16)] = carry[4 + k]
        pltpu.sync_copy(stat_v, stats_hbm.at[c, s])

    @pl.when(c == 0)
    def _():
        run_half(ts0, eg0, 0, m0_hbm)

    @pl.when(c == 1)
    def _():
        run_half(ts1, eg1, H, m1_hbm)

    plsc.subcore_barrier()

    @pl.when((s == 0) & (c == 0))
    def _():
        pltpu.sync_copy(acc_sh, acc0_hbm)

    @pl.when((s == 0) & (c == 1))
    def _():
        pltpu.sync_copy(acc_sh, acc1_hbm)


def _edge_epi_body(m0, m1, ef, stats, gamma, beta, y):
    st = stats[...]
    red = jnp.sum(st, axis=1)
    sum_m = jnp.concatenate([red[0:1, 0:H], red[1:2, 0:H]], axis=1)
    sum_q = jnp.concatenate([red[0:1, H:], red[1:2, H:]], axis=1)
    mu = sum_m * (1.0 / E)
    var = sum_q * (1.0 / E) - mu * mu
    m = jnp.concatenate([m0[...], m1[...]], axis=1)
    t = gamma[...] * (m - mu) * lax.rsqrt(var + 1e-5) + beta[...]
    y[...] = ef[...] + t * (1.0 / (1.0 + jnp.exp(-t)))


def _node_epi_body(acc0, acc1, cx, nf, gamma, beta, x):
    a0 = acc0[...]
    a1 = acc1[...]
    num = jnp.concatenate([a0[:, :H], a1[:, :H]], axis=1)
    den = jnp.concatenate([a0[:, H:], a1[:, H:]], axis=1)
    v = cx[...] + num / (den + 1e-6)
    mu = jnp.mean(v, axis=0, keepdims=True)
    var = jnp.mean(v * v, axis=0, keepdims=True) - mu * mu
    t = gamma[...] * (v - mu) * lax.rsqrt(var + 1e-5) + beta[...]
    x[...] = nf[...] + t * (1.0 / (1.0 + jnp.exp(-t)))


def kernel(node_feats, edge_feats, edge_index, W_src_gate, b_src_gate,
           W_dst_gate, b_dst_gate, W_edge_gate, b_edge_gate,
           W_dst_update, b_dst_update, W_src_update, b_src_update,
           bn_nodes_gamma, bn_nodes_beta, bn_edges_gamma, bn_edges_beta):
    src = edge_index[0]
    dst = edge_index[1]
    f32 = jnp.float32

    nb = 1000
    bdu = b_dst_update.reshape(1, D)
    bsu = b_src_update.reshape(1, D)
    ts0, ts1, e_dst, Cx = pl.pallas_call(
        _node_pre_body,
        grid=(N // nb,),
        in_specs=[
            pl.BlockSpec((nb, D), lambda i: (i, 0)),
            pl.BlockSpec((D, D), lambda i: (0, 0)),
            pl.BlockSpec((D, D), lambda i: (0, 0)),
            pl.BlockSpec((D, D), lambda i: (0, 0)),
            pl.BlockSpec((D, D), lambda i: (0, 0)),
            pl.BlockSpec((1, D), lambda i: (0, 0)),
            pl.BlockSpec((1, D), lambda i: (0, 0)),
        ],
        out_specs=[pl.BlockSpec((nb, D), lambda i: (i, 0))] * 4,
        out_shape=[jax.ShapeDtypeStruct((N, D), f32)] * 4,
    )(node_feats, W_src_gate, W_dst_gate, W_dst_update, W_src_update,
      bdu, bsu)

    eb = 4000
    gate_bias = (b_edge_gate + b_src_gate + b_dst_gate).reshape(1, D)
    eg0, eg1 = pl.pallas_call(
        _gate_body,
        grid=(E // eb,),
        in_specs=[
            pl.BlockSpec((eb, D), lambda i: (i, 0)),
            pl.BlockSpec((D, D), lambda i: (0, 0)),
            pl.BlockSpec((1, D), lambda i: (0, 0)),
        ],
        out_specs=[pl.BlockSpec((eb, H), lambda i: (i, 0))] * 2,
        out_shape=[jax.ShapeDtypeStruct((E, H), f32)] * 2,
    )(edge_feats, W_edge_gate, gate_bias)

    zer = jnp.zeros((N, D), f32)

    mesh = plsc.VectorSubcoreMesh(core_axis_name="c", subcore_axis_name="s")
    sc_fn = pl.kernel(
        _sc_body,
        out_type=[
            jax.ShapeDtypeStruct((E, H), f32),
            jax.ShapeDtypeStruct((E, H), f32),
            jax.ShapeDtypeStruct((N, D), f32),
            jax.ShapeDtypeStruct((N, D), f32),
            jax.ShapeDtypeStruct((2, NT, D), f32),
        ],
        mesh=mesh,
        scratch_types=(
            [pltpu.VMEM((C,), jnp.int32)] * 4 +
            [pltpu.VMEM((C, D), f32)] * 2 +
            [pltpu.VMEM((C, D), f32)] * 2 +
            [pltpu.VMEM((C, H), f32)] * 2 +
            [pltpu.VMEM((C, D), f32)] * 2 +
            [pltpu.VMEM((C,), jnp.int32)] * 2 +
            [pltpu.VMEM((D,), f32),
             pltpu.VMEM_SHARED((N, D), f32)] +
            [pltpu.SemaphoreType.DMA] * 14
        ),
    )
    m0, m1, acc0, acc1, stats = sc_fn(src, dst, ts0, ts1, e_dst,
                                      eg0, eg1, zer)

    y = pl.pallas_call(
        _edge_epi_body,
        grid=(E // eb,),
        in_specs=[
            pl.BlockSpec((eb, H), lambda i: (i, 0)),
            pl.BlockSpec((eb, H), lambda i: (i, 0)),
            pl.BlockSpec((eb, D), lambda i: (i, 0)),
            pl.BlockSpec((2, NT, D), lambda i: (0, 0, 0)),
            pl.BlockSpec((1, D), lambda i: (0, 0)),
            pl.BlockSpec((1, D), lambda i: (0, 0)),
        ],
        out_specs=pl.BlockSpec((eb, D), lambda i: (i, 0)),
        out_shape=jax.ShapeDtypeStruct((E, D), f32),
    )(m0, m1, edge_feats, stats, bn_edges_gamma.reshape(1, D),
      bn_edges_beta.reshape(1, D))

    x = pl.pallas_call(
        _node_epi_body,
        in_specs=[pl.BlockSpec((N, D), lambda: (0, 0))] * 4 +
                 [pl.BlockSpec((1, D), lambda: (0, 0))] * 2,
        out_specs=pl.BlockSpec((N, D), lambda: (0, 0)),
        out_shape=jax.ShapeDtypeStruct((N, D), f32),
    )(acc0, acc1, Cx, node_feats, bn_nodes_gamma.reshape(1, D),
      bn_nodes_beta.reshape(1, D))

    return (x, y)
